# Initial kernel scaffold; baseline (speedup 1.0000x reference)
import jax
import jax.numpy as jnp
from jax.experimental import pallas as pl


def kernel(x, edge_index, W1_rel, b1, W1_root, W2_rel, b2, W2_root):
    raise NotImplementedError("write your pallas kernel here")


# speedup vs baseline: 6.0077x; 6.0077x over previous
import functools

import jax
import jax.numpy as jnp
from jax import lax
from jax.experimental import pallas as pl
from jax.experimental.pallas import tpu as pltpu
from jax.experimental.pallas import tpu_sc as plsc

NC = 2
NS = 16


def _make_seg_sum(n_nodes, n_pad, n_feat, n_edges, chunk):
    workers = NC * NS
    edges_per_tile = n_edges // workers
    n_chunks = edges_per_tile // chunk
    rows_per_tile = n_pad // NS

    mesh = plsc.VectorSubcoreMesh(core_axis_name="c", subcore_axis_name="s")

    @functools.partial(
        pl.kernel,
        out_type=jax.ShapeDtypeStruct((NC, n_pad, n_feat), jnp.float32),
        mesh=mesh,
        scratch_types=[
            pltpu.VMEM_SHARED((n_pad, n_feat), jnp.float32),
            pltpu.VMEM((chunk,), jnp.int32),
            pltpu.VMEM((chunk,), jnp.int32),
            pltpu.VMEM((chunk, n_feat), jnp.float32),
            pltpu.SemaphoreType.DMA,
        ],
    )
    def seg_sum(x_hbm, src_hbm, dst_hbm, init_hbm, out_hbm,
                acc, src_v, dst_v, rows_v, sem):
        c = lax.axis_index("c")
        s = lax.axis_index("s")
        w = s * NC + c
        r0 = pl.multiple_of(s * rows_per_tile, 8)
        pltpu.sync_copy(init_hbm.at[c, pl.ds(r0, rows_per_tile)],
                        acc.at[pl.ds(r0, rows_per_tile)])
        base = w * edges_per_tile
        plsc.subcore_barrier()

        @pl.loop(0, n_chunks)
        def _(j):
            off = pl.multiple_of(base + j * chunk, 8)
            pltpu.sync_copy(src_hbm.at[pl.ds(off, chunk)], src_v)
            pltpu.sync_copy(dst_hbm.at[pl.ds(off, chunk)], dst_v)
            pltpu.async_copy(x_hbm.at[src_v], rows_v, sem).wait()
            pltpu.sync_copy(rows_v, acc.at[dst_v], add=True)

        plsc.subcore_barrier()
        pltpu.sync_copy(acc.at[pl.ds(r0, rows_per_tile)],
                        out_hbm.at[c, pl.ds(r0, rows_per_tile)])

    return seg_sum


def _dense_body(agg_ref, x_ref, w1a_ref, w1b_ref, b1_ref, w2a_ref, w2b_ref,
                b2_ref, p_ref, q_ref):
    agg = agg_ref[0] + agg_ref[1]
    h = jnp.dot(agg, w1a_ref[...], preferred_element_type=jnp.float32)
    h = h + jnp.dot(x_ref[...], w1b_ref[...], preferred_element_type=jnp.float32)
    h = jnp.maximum(h + b1_ref[...], 0.0)
    p_ref[...] = jnp.dot(h, w2a_ref[...], preferred_element_type=jnp.float32)
    q_ref[...] = (jnp.dot(h, w2b_ref[...], preferred_element_type=jnp.float32)
                  + b2_ref[...])


def _combine_body(parts_ref, out_ref):
    out_ref[...] = parts_ref[0] + parts_ref[1]


def kernel(x, edge_index, W1_rel, b1, W1_root, W2_rel, b2, W2_root):
    n_nodes, d_feat = x.shape
    n_edges = edge_index.shape[1]
    d_hid = W1_rel.shape[0]
    n_cls = W2_rel.shape[0]
    cls_pad = 128

    ei = edge_index.astype(jnp.int32)
    src, dst = ei[0], ei[1]
    n_pad = ((n_nodes + 127) // 128) * 128

    seg1 = _make_seg_sum(n_nodes, n_pad, d_feat, n_edges, chunk=80)
    init1 = jnp.zeros((NC, n_pad, d_feat), jnp.float32)
    agg1 = seg1(x, src, dst, init1)

    w1a = W1_rel.T
    w1b = W1_root.T
    w2a = jnp.zeros((d_hid, cls_pad), jnp.float32).at[:, :n_cls].set(W2_rel.T)
    w2b = jnp.zeros((d_hid, cls_pad), jnp.float32).at[:, :n_cls].set(W2_root.T)
    b2p = jnp.zeros((1, cls_pad), jnp.float32).at[0, :n_cls].set(b2)

    tn = 1000
    grid = (n_nodes // tn,)
    p, q = pl.pallas_call(
        _dense_body,
        grid=grid,
        in_specs=[
            pl.BlockSpec((NC, tn, d_feat), lambda i: (0, i, 0)),
            pl.BlockSpec((tn, d_feat), lambda i: (i, 0)),
            pl.BlockSpec((d_feat, d_hid), lambda i: (0, 0)),
            pl.BlockSpec((d_feat, d_hid), lambda i: (0, 0)),
            pl.BlockSpec((1, d_hid), lambda i: (0, 0)),
            pl.BlockSpec((d_hid, cls_pad), lambda i: (0, 0)),
            pl.BlockSpec((d_hid, cls_pad), lambda i: (0, 0)),
            pl.BlockSpec((1, cls_pad), lambda i: (0, 0)),
        ],
        out_specs=[
            pl.BlockSpec((tn, cls_pad), lambda i: (i, 0)),
            pl.BlockSpec((tn, cls_pad), lambda i: (i, 0)),
        ],
        out_shape=[
            jax.ShapeDtypeStruct((n_nodes, cls_pad), jnp.float32),
            jax.ShapeDtypeStruct((n_nodes, cls_pad), jnp.float32),
        ],
    )(agg1, x, w1a, w1b, b1.reshape(1, -1), w2a, w2b, b2p)

    seg2 = _make_seg_sum(n_nodes, n_pad, cls_pad, n_edges, chunk=80)
    init2 = jnp.zeros((NC, n_pad, cls_pad), jnp.float32).at[0, :n_nodes].set(q)
    agg2 = seg2(p, src, dst, init2)

    out = pl.pallas_call(
        _combine_body,
        grid=grid,
        in_specs=[pl.BlockSpec((NC, tn, cls_pad), lambda i: (0, i, 0))],
        out_specs=pl.BlockSpec((tn, cls_pad), lambda i: (i, 0)),
        out_shape=jax.ShapeDtypeStruct((n_nodes, cls_pad), jnp.float32),
    )(agg2)
    return out[:, :n_cls]
